# baseline (speedup 1.0000x reference)
import functools

import jax
import jax.numpy as jnp
from jax import lax
from jax.experimental import pallas as pl
from jax.experimental.pallas import tpu as pltpu
from jax.experimental.pallas import tpu_sc as plsc

B, N1, N2, C1, C2, K = 8, 4096, 1024, 32, 32, 16
TN = 512
QT = 512
ROWS = B * N1 * K
MKE = float(ROWS)
M3 = float(B * N1)
F32 = jnp.float32
HIGH = lax.Precision.HIGHEST

NW = 32
ROWS_B = N1 * K
RPW = ROWS_B // NW
CH = 256
NCH = RPW // CH


def _s0_body(xyz1_ref, xyz2_ref, feat2_ref, W1_ref, b1_ref, t2_ref, q_ref):
    W1 = W1_ref[...]
    Wf = W1[:, :C2]
    Wx = W1[:, C2:]
    f2 = feat2_ref[0]
    x2 = xyz2_ref[0]
    t2 = (lax.dot_general(f2, Wf, (((1,), (1,)), ((), ())), precision=HIGH)
          + lax.dot_general(x2, Wx, (((1,), (1,)), ((), ())), precision=HIGH))
    t2_ref[0] = jnp.concatenate([t2, jnp.zeros((N2, 64), F32)], axis=1)
    x1 = xyz1_ref[0]
    q_ref[0] = b1_ref[...] - lax.dot_general(x1, Wx, (((1,), (1,)), ((), ())),
                                             precision=HIGH)


def _s1_body(xyz1_ref, xyz2_ref, idx_ref, d_scr, i_scr):
    x1 = xyz1_ref[...]
    x2 = xyz2_ref[...]
    n1 = jnp.sum(x1 * x1, axis=1)
    n2 = jnp.sum(x2 * x2, axis=1)
    dd = lax.dot_general(x2.astype(jnp.bfloat16), x1.astype(jnp.bfloat16),
                         (((1,), (1,)), ((), ())),
                         preferred_element_type=F32)
    d_scr[...] = (-2.0 * dd + n1[None, :]) + n2[:, None]
    i_scr[...] = lax.broadcasted_iota(jnp.int32, (N2, TN), 0).astype(F32)
    INF = jnp.float32(jnp.inf)
    BIGI = jnp.float32(2 ** 30)
    for k in range(K):
        v = d_scr[...]
        mn = jnp.min(v, axis=0)
        eq = v == mn[None, :]
        ami = jnp.min(jnp.where(eq, i_scr[...], BIGI), axis=0)
        idx_ref[0, k, :] = ami.astype(jnp.int32)
        d_scr[...] = jnp.where(eq, INF, v)


def _gather(t2f, gidx2d):
    mesh = plsc.VectorSubcoreMesh(core_axis_name="c", subcore_axis_name="s")

    @functools.partial(
        pl.kernel,
        out_type=jax.ShapeDtypeStruct((ROWS_B, 128), F32),
        mesh=mesh,
        scratch_types=[
            pltpu.VMEM((RPW // 128, 128), jnp.int32),
            pltpu.VMEM((CH, 128), F32),
            pltpu.VMEM((CH, 128), F32),
            pltpu.SemaphoreType.DMA,
            pltpu.SemaphoreType.DMA,
        ],
    )
    def k(t2_hbm, gidx_hbm, out_hbm, idx_v, rows_a, rows_b, gsem, wsem):
        wid = lax.axis_index("s") * 2 + lax.axis_index("c")
        nj = CH // 128
        bufs = (rows_a, rows_b)
        pltpu.sync_copy(gidx_hbm.at[pl.ds(wid * (RPW // 128), RPW // 128)],
                        idx_v)
        wcps = [None, None]
        prev = None
        for i in range(NCH):
            p = i & 1
            if wcps[p] is not None:
                for c in wcps[p]:
                    c.wait()
                wcps[p] = None
            g = [pltpu.async_copy(t2_hbm.at[idx_v.at[i * nj + j]],
                                  bufs[p].at[pl.ds(j * 128, 128)], gsem)
                 for j in range(nj)]
            if prev is not None:
                pp, pg = prev
                for c in pg:
                    c.wait()
                rowbase = wid * RPW + (i - 1) * CH
                wcps[pp] = [pltpu.async_copy(
                    bufs[pp], out_hbm.at[pl.ds(rowbase, CH)], wsem)]
            prev = (p, g)
        pp, pg = prev
        for c in pg:
            c.wait()
        pltpu.sync_copy(bufs[pp],
                        out_hbm.at[pl.ds(wid * RPW + (NCH - 1) * CH, CH)])
        for w in wcps:
            if w is not None:
                for c in w:
                    c.wait()

    return k(t2f, gidx2d)


def _s3_body(G_ref, q_ref, st_ref):
    i = pl.program_id(0)
    y1 = G_ref[:, :, :64] + q_ref[...][:, None, :]
    s = jnp.sum(y1, axis=(0, 1))
    ss = jnp.sum(y1 * y1, axis=(0, 1))
    acc = jnp.broadcast_to(jnp.concatenate([s, ss])[None, :], (8, 128))

    @pl.when(i == 0)
    def _():
        st_ref[...] = jnp.zeros((8, 128), F32)

    st_ref[...] += acc


def _dot3(x, w):
    xh = x.astype(jnp.bfloat16)
    xl = (x - xh.astype(F32)).astype(jnp.bfloat16)
    wh = w.astype(jnp.bfloat16)
    wl = (w - wh.astype(F32)).astype(jnp.bfloat16)
    dn = (((1,), (1,)), ((), ()))
    return (lax.dot_general(xh, wh, dn, preferred_element_type=F32)
            + (lax.dot_general(xh, wl, dn, preferred_element_type=F32)
               + lax.dot_general(xl, wh, dn, preferred_element_type=F32)))


def _bn(y, st, mcount, g, be):
    mu = st[:64] / mcount
    var = st[64:] / mcount - mu * mu
    rstd = lax.rsqrt(var + 1e-5)
    return jnp.maximum((y - mu) * rstd * g + be, 0.0)


def _s4_body(G_ref, q_ref, st1_ref, W2_ref, b2_ref, g1_ref, be1_ref, st_ref):
    i = pl.program_id(0)
    y1 = G_ref[:, :, :64] + q_ref[...][:, None, :]
    h1 = _bn(y1, st1_ref[0, :], MKE, g1_ref[0], be1_ref[0])
    h1f = h1.reshape(QT * K, 64)
    y2 = _dot3(h1f, W2_ref[...]) + b2_ref[0]
    s = jnp.sum(y2, axis=0)
    ss = jnp.sum(y2 * y2, axis=0)
    acc = jnp.broadcast_to(jnp.concatenate([s, ss])[None, :], (8, 128))

    @pl.when(i == 0)
    def _():
        st_ref[...] = jnp.zeros((8, 128), F32)

    st_ref[...] += acc


def _s5_body(G_ref, q_ref, f1_ref, st1_ref, st2_ref, W2_ref, b2_ref,
             g1_ref, be1_ref, g2_ref, be2_ref, W3_ref, b3_ref,
             y3_ref, st_ref):
    i = pl.program_id(0)
    y1 = G_ref[:, :, :64] + q_ref[...][:, None, :]
    h1 = _bn(y1, st1_ref[0, :], MKE, g1_ref[0], be1_ref[0])
    h1f = h1.reshape(QT * K, 64)
    y2 = _dot3(h1f, W2_ref[...]) + b2_ref[0]
    h2 = _bn(y2, st2_ref[0, :], MKE, g2_ref[0], be2_ref[0])
    mp = jnp.max(h2.reshape(QT, K, 64), axis=1)
    W3 = W3_ref[...]
    y3 = _dot3(mp, W3[:, :64]) + _dot3(f1_ref[...], W3[:, 64:]) + b3_ref[0]
    y3_ref[...] = y3
    s = jnp.sum(y3, axis=0)
    ss = jnp.sum(y3 * y3, axis=0)
    acc = jnp.broadcast_to(jnp.concatenate([s, ss])[None, :], (8, 128))

    @pl.when(i == 0)
    def _():
        st_ref[...] = jnp.zeros((8, 128), F32)

    st_ref[...] += acc


def _s6_body(y3_ref, st3_ref, g3_ref, be3_ref, o_ref):
    o_ref[...] = _bn(y3_ref[...], st3_ref[0, :], M3, g3_ref[0], be3_ref[0])


def kernel(xyz1, xyz2, feat1, feat2, W1, b1, g1, be1, W2, b2, g2, be2,
           W3, b3, g3, be3):
    b1r, g1r, be1r = b1.reshape(1, 64), g1.reshape(1, 64), be1.reshape(1, 64)
    b2r, g2r, be2r = b2.reshape(1, 64), g2.reshape(1, 64), be2.reshape(1, 64)
    b3r, g3r, be3r = b3.reshape(1, 64), g3.reshape(1, 64), be3.reshape(1, 64)

    t2, q = pl.pallas_call(
        _s0_body,
        grid=(B,),
        in_specs=[
            pl.BlockSpec((1, N1, 3), lambda b: (b, 0, 0)),
            pl.BlockSpec((1, N2, 3), lambda b: (b, 0, 0)),
            pl.BlockSpec((1, N2, C2), lambda b: (b, 0, 0)),
            pl.BlockSpec((64, 35), lambda b: (0, 0)),
            pl.BlockSpec((1, 64), lambda b: (0, 0)),
        ],
        out_specs=[
            pl.BlockSpec((1, N2, 128), lambda b: (b, 0, 0)),
            pl.BlockSpec((1, N1, 64), lambda b: (b, 0, 0)),
        ],
        out_shape=[
            jax.ShapeDtypeStruct((B, N2, 128), F32),
            jax.ShapeDtypeStruct((B, N1, 64), F32),
        ],
    )(xyz1, xyz2, feat2, W1, b1r)

    n1t = N1 // TN
    s1_call = pl.pallas_call(
        _s1_body,
        grid=(n1t,),
        in_specs=[
            pl.BlockSpec((TN, 3), lambda t: (t, 0)),
            pl.BlockSpec((N2, 3), lambda t: (0, 0)),
        ],
        out_specs=pl.BlockSpec((1, K, TN), lambda t: (t, 0, 0)),
        out_shape=jax.ShapeDtypeStruct((n1t, K, TN), jnp.int32),
        scratch_shapes=[pltpu.VMEM((N2, TN), F32), pltpu.VMEM((N2, TN), F32)],
    )

    Gs = []
    for b in range(B):
        idx3 = s1_call(xyz1[b], xyz2[b])
        gidx2d = jnp.transpose(idx3, (0, 2, 1)).reshape(ROWS_B // 128, 128)
        Gs.append(_gather(t2[b], gidx2d).reshape(N1, K, 128))

    nq = N1 // QT
    stspec = pl.BlockSpec((8, 128), lambda i: (0, 0))
    stshape = jax.ShapeDtypeStruct((8, 128), F32)
    gspec = pl.BlockSpec((QT, K, 128), lambda i: (i, 0, 0))
    qspec = pl.BlockSpec((QT, 64), lambda i: (i, 0))
    vspec = pl.BlockSpec((1, 64), lambda i: (0, 0))

    s3_call = pl.pallas_call(
        _s3_body,
        grid=(nq,),
        in_specs=[gspec, qspec],
        out_specs=stspec,
        out_shape=stshape,
    )
    qf = q.reshape(B, N1, 64)
    st1 = sum(s3_call(Gs[b], qf[b]) for b in range(B))

    s4_call = pl.pallas_call(
        _s4_body,
        grid=(nq,),
        in_specs=[gspec, qspec, stspec,
                  pl.BlockSpec((64, 64), lambda i: (0, 0)), vspec, vspec,
                  vspec],
        out_specs=stspec,
        out_shape=stshape,
    )
    st2 = sum(s4_call(Gs[b], qf[b], st1, W2, b2r, g1r, be1r)
              for b in range(B))

    s5_call = pl.pallas_call(
        _s5_body,
        grid=(nq,),
        in_specs=[gspec, qspec, pl.BlockSpec((QT, C1), lambda i: (i, 0)),
                  stspec, stspec,
                  pl.BlockSpec((64, 64), lambda i: (0, 0)), vspec, vspec,
                  vspec, vspec, vspec,
                  pl.BlockSpec((64, 96), lambda i: (0, 0)), vspec],
        out_specs=[qspec, stspec],
        out_shape=[jax.ShapeDtypeStruct((N1, 64), F32), stshape],
    )
    y3s = []
    st3 = None
    for b in range(B):
        y3b, st3b = s5_call(Gs[b], qf[b], feat1[b], st1, st2, W2, b2r,
                            g1r, be1r, g2r, be2r, W3, b3r)
        y3s.append(y3b)
        st3 = st3b if st3 is None else st3 + st3b

    s6_call = pl.pallas_call(
        _s6_body,
        grid=(nq,),
        in_specs=[qspec, stspec, vspec, vspec],
        out_specs=qspec,
        out_shape=jax.ShapeDtypeStruct((N1, 64), F32),
    )
    return jnp.stack([s6_call(y3s[b], st3, g3r, be3r) for b in range(B)])


# speedup vs baseline: 1.0624x; 1.0624x over previous
import functools

import jax
import jax.numpy as jnp
from jax import lax
from jax.experimental import pallas as pl
from jax.experimental.pallas import tpu as pltpu
from jax.experimental.pallas import tpu_sc as plsc

B, N1, N2, C1, C2, K = 8, 4096, 1024, 32, 32, 16
TN = 512
QT = 512
ROWS = B * N1 * K
MKE = float(ROWS)
M3 = float(B * N1)
F32 = jnp.float32
HIGH = lax.Precision.HIGHEST

NW = 32
ROWS_B = N1 * K
RPW = ROWS_B // NW
CH = 256
NCH = RPW // CH


def _s0_body(xyz1_ref, xyz2_ref, feat2_ref, W1_ref, b1_ref, t2_ref, q_ref):
    W1 = W1_ref[...]
    Wf = W1[:, :C2]
    Wx = W1[:, C2:]
    f2 = feat2_ref[0]
    x2 = xyz2_ref[0]
    t2 = (lax.dot_general(f2, Wf, (((1,), (1,)), ((), ())), precision=HIGH)
          + lax.dot_general(x2, Wx, (((1,), (1,)), ((), ())), precision=HIGH))
    t2_ref[0] = jnp.concatenate([t2, jnp.zeros((N2, 64), F32)], axis=1)
    x1 = xyz1_ref[0]
    q_ref[0] = b1_ref[...] - lax.dot_general(x1, Wx, (((1,), (1,)), ((), ())),
                                             precision=HIGH)


def _s1_body(xyz1_ref, xyz2_ref, idx_ref, d_scr, i_scr):
    x1 = xyz1_ref[...]
    x2 = xyz2_ref[...]
    n1 = jnp.sum(x1 * x1, axis=1)
    n2 = jnp.sum(x2 * x2, axis=1)
    dd = lax.dot_general(x2.astype(jnp.bfloat16), x1.astype(jnp.bfloat16),
                         (((1,), (1,)), ((), ())),
                         preferred_element_type=F32)
    d_scr[...] = (-2.0 * dd + n1[None, :]) + n2[:, None]
    i_scr[...] = lax.broadcasted_iota(jnp.int32, (N2, TN), 0).astype(F32)
    INF = jnp.float32(jnp.inf)
    BIGI = jnp.float32(2 ** 30)
    for k in range(K):
        v = d_scr[...]
        mn = jnp.min(v, axis=0)
        eq = v == mn[None, :]
        ami = jnp.min(jnp.where(eq, i_scr[...], BIGI), axis=0)
        idx_ref[0, k, :] = ami.astype(jnp.int32)
        d_scr[...] = jnp.where(eq, INF, v)


def _gather(t2f, gidx2d):
    mesh = plsc.VectorSubcoreMesh(core_axis_name="c", subcore_axis_name="s")

    @functools.partial(
        pl.kernel,
        out_type=jax.ShapeDtypeStruct((ROWS_B, 128), F32),
        mesh=mesh,
        scratch_types=[
            pltpu.VMEM((RPW // 128, 128), jnp.int32),
            pltpu.VMEM((CH, 128), F32),
            pltpu.VMEM((CH, 128), F32),
            pltpu.SemaphoreType.DMA,
            pltpu.SemaphoreType.DMA,
        ],
    )
    def k(t2_hbm, gidx_hbm, out_hbm, idx_v, rows_a, rows_b, gsem, wsem):
        wid = lax.axis_index("s") * 2 + lax.axis_index("c")
        nj = CH // 128
        bufs = (rows_a, rows_b)
        pltpu.sync_copy(gidx_hbm.at[pl.ds(wid * (RPW // 128), RPW // 128)],
                        idx_v)
        wcps = [None, None]
        prev = None
        for i in range(NCH):
            p = i & 1
            if wcps[p] is not None:
                for c in wcps[p]:
                    c.wait()
                wcps[p] = None
            g = [pltpu.async_copy(t2_hbm.at[idx_v.at[i * nj + j]],
                                  bufs[p].at[pl.ds(j * 128, 128)], gsem)
                 for j in range(nj)]
            if prev is not None:
                pp, pg = prev
                for c in pg:
                    c.wait()
                rowbase = wid * RPW + (i - 1) * CH
                wcps[pp] = [pltpu.async_copy(
                    bufs[pp], out_hbm.at[pl.ds(rowbase, CH)], wsem)]
            prev = (p, g)
        pp, pg = prev
        for c in pg:
            c.wait()
        pltpu.sync_copy(bufs[pp],
                        out_hbm.at[pl.ds(wid * RPW + (NCH - 1) * CH, CH)])
        for w in wcps:
            if w is not None:
                for c in w:
                    c.wait()

    return k(t2f, gidx2d)


def _s3_body(G_ref, q_ref, st_ref):
    i = pl.program_id(0)
    y1 = G_ref[0, :, :, :64] + q_ref[...][None, :, :]
    s = jnp.sum(y1, axis=(0, 1))
    ss = jnp.sum(y1 * y1, axis=(0, 1))
    acc = jnp.broadcast_to(jnp.concatenate([s, ss])[None, :], (8, 128))

    @pl.when(i == 0)
    def _():
        st_ref[...] = jnp.zeros((8, 128), F32)

    st_ref[...] += acc


def _dot3(x, w):
    xh = x.astype(jnp.bfloat16)
    xl = (x - xh.astype(F32)).astype(jnp.bfloat16)
    wh = w.astype(jnp.bfloat16)
    wl = (w - wh.astype(F32)).astype(jnp.bfloat16)
    dn = (((1,), (1,)), ((), ()))
    return (lax.dot_general(xh, wh, dn, preferred_element_type=F32)
            + (lax.dot_general(xh, wl, dn, preferred_element_type=F32)
               + lax.dot_general(xl, wh, dn, preferred_element_type=F32)))


def _bn(y, st, mcount, g, be):
    mu = st[:64] / mcount
    var = st[64:] / mcount - mu * mu
    rstd = lax.rsqrt(var + 1e-5)
    return jnp.maximum((y - mu) * rstd * g + be, 0.0)


def _s4_body(G_ref, q_ref, st1_ref, W2_ref, b2_ref, g1_ref, be1_ref, st_ref):
    i = pl.program_id(0)
    y1 = G_ref[0, :, :, :64] + q_ref[...][None, :, :]
    h1 = _bn(y1, st1_ref[0, :], MKE, g1_ref[0], be1_ref[0])
    h1f = h1.reshape(K * QT, 64)
    y2 = _dot3(h1f, W2_ref[...]) + b2_ref[0]
    s = jnp.sum(y2, axis=0)
    ss = jnp.sum(y2 * y2, axis=0)
    acc = jnp.broadcast_to(jnp.concatenate([s, ss])[None, :], (8, 128))

    @pl.when(i == 0)
    def _():
        st_ref[...] = jnp.zeros((8, 128), F32)

    st_ref[...] += acc


def _s5_body(G_ref, q_ref, f1_ref, st1_ref, st2_ref, W2_ref, b2_ref,
             g1_ref, be1_ref, g2_ref, be2_ref, W3_ref, b3_ref,
             y3_ref, st_ref):
    i = pl.program_id(0)
    y1 = G_ref[0, :, :, :64] + q_ref[...][None, :, :]
    h1 = _bn(y1, st1_ref[0, :], MKE, g1_ref[0], be1_ref[0])
    h1f = h1.reshape(K * QT, 64)
    y2 = _dot3(h1f, W2_ref[...]) + b2_ref[0]
    h2 = _bn(y2, st2_ref[0, :], MKE, g2_ref[0], be2_ref[0])
    mp = jnp.max(h2.reshape(K, QT, 64), axis=0)
    W3 = W3_ref[...]
    y3 = _dot3(mp, W3[:, :64]) + _dot3(f1_ref[...], W3[:, 64:]) + b3_ref[0]
    y3_ref[...] = y3
    s = jnp.sum(y3, axis=0)
    ss = jnp.sum(y3 * y3, axis=0)
    acc = jnp.broadcast_to(jnp.concatenate([s, ss])[None, :], (8, 128))

    @pl.when(i == 0)
    def _():
        st_ref[...] = jnp.zeros((8, 128), F32)

    st_ref[...] += acc


def _s6_body(y3_ref, st3_ref, g3_ref, be3_ref, o_ref):
    o_ref[...] = _bn(y3_ref[...], st3_ref[0, :], M3, g3_ref[0], be3_ref[0])


def kernel(xyz1, xyz2, feat1, feat2, W1, b1, g1, be1, W2, b2, g2, be2,
           W3, b3, g3, be3):
    b1r, g1r, be1r = b1.reshape(1, 64), g1.reshape(1, 64), be1.reshape(1, 64)
    b2r, g2r, be2r = b2.reshape(1, 64), g2.reshape(1, 64), be2.reshape(1, 64)
    b3r, g3r, be3r = b3.reshape(1, 64), g3.reshape(1, 64), be3.reshape(1, 64)

    t2, q = pl.pallas_call(
        _s0_body,
        grid=(B,),
        in_specs=[
            pl.BlockSpec((1, N1, 3), lambda b: (b, 0, 0)),
            pl.BlockSpec((1, N2, 3), lambda b: (b, 0, 0)),
            pl.BlockSpec((1, N2, C2), lambda b: (b, 0, 0)),
            pl.BlockSpec((64, 35), lambda b: (0, 0)),
            pl.BlockSpec((1, 64), lambda b: (0, 0)),
        ],
        out_specs=[
            pl.BlockSpec((1, N2, 128), lambda b: (b, 0, 0)),
            pl.BlockSpec((1, N1, 64), lambda b: (b, 0, 0)),
        ],
        out_shape=[
            jax.ShapeDtypeStruct((B, N2, 128), F32),
            jax.ShapeDtypeStruct((B, N1, 64), F32),
        ],
    )(xyz1, xyz2, feat2, W1, b1r)

    n1t = N1 // TN
    s1_call = pl.pallas_call(
        _s1_body,
        grid=(n1t,),
        in_specs=[
            pl.BlockSpec((TN, 3), lambda t: (t, 0)),
            pl.BlockSpec((N2, 3), lambda t: (0, 0)),
        ],
        out_specs=pl.BlockSpec((1, K, TN), lambda t: (t, 0, 0)),
        out_shape=jax.ShapeDtypeStruct((n1t, K, TN), jnp.int32),
        scratch_shapes=[pltpu.VMEM((N2, TN), F32), pltpu.VMEM((N2, TN), F32)],
    )

    Gs = []
    for b in range(B):
        idx3 = s1_call(xyz1[b], xyz2[b])
        gidx2d = idx3.reshape(ROWS_B // 128, 128)
        Gs.append(_gather(t2[b], gidx2d).reshape(N1 // TN, K, TN, 128))

    nq = N1 // QT
    stspec = pl.BlockSpec((8, 128), lambda i: (0, 0))
    stshape = jax.ShapeDtypeStruct((8, 128), F32)
    gspec = pl.BlockSpec((1, K, QT, 128), lambda i: (i, 0, 0, 0))
    qspec = pl.BlockSpec((QT, 64), lambda i: (i, 0))
    vspec = pl.BlockSpec((1, 64), lambda i: (0, 0))

    s3_call = pl.pallas_call(
        _s3_body,
        grid=(nq,),
        in_specs=[gspec, qspec],
        out_specs=stspec,
        out_shape=stshape,
    )
    qf = q.reshape(B, N1, 64)
    st1 = sum(s3_call(Gs[b], qf[b]) for b in range(B))

    s4_call = pl.pallas_call(
        _s4_body,
        grid=(nq,),
        in_specs=[gspec, qspec, stspec,
                  pl.BlockSpec((64, 64), lambda i: (0, 0)), vspec, vspec,
                  vspec],
        out_specs=stspec,
        out_shape=stshape,
    )
    st2 = sum(s4_call(Gs[b], qf[b], st1, W2, b2r, g1r, be1r)
              for b in range(B))

    s5_call = pl.pallas_call(
        _s5_body,
        grid=(nq,),
        in_specs=[gspec, qspec, pl.BlockSpec((QT, C1), lambda i: (i, 0)),
                  stspec, stspec,
                  pl.BlockSpec((64, 64), lambda i: (0, 0)), vspec, vspec,
                  vspec, vspec, vspec,
                  pl.BlockSpec((64, 96), lambda i: (0, 0)), vspec],
        out_specs=[qspec, stspec],
        out_shape=[jax.ShapeDtypeStruct((N1, 64), F32), stshape],
    )
    y3s = []
    st3 = None
    for b in range(B):
        y3b, st3b = s5_call(Gs[b], qf[b], feat1[b], st1, st2, W2, b2r,
                            g1r, be1r, g2r, be2r, W3, b3r)
        y3s.append(y3b)
        st3 = st3b if st3 is None else st3 + st3b

    s6_call = pl.pallas_call(
        _s6_body,
        grid=(nq,),
        in_specs=[qspec, stspec, vspec, vspec],
        out_specs=qspec,
        out_shape=jax.ShapeDtypeStruct((N1, 64), F32),
    )
    return jnp.stack([s6_call(y3s[b], st3, g3r, be3r) for b in range(B)])
